# baseline (speedup 1.0000x reference)
import jax
import jax.numpy as jnp
from jax import lax
from jax.experimental import pallas as pl
from jax.experimental.pallas import tpu as pltpu
from jax.experimental.pallas import tpu_sc as plsc

N = 10000
NP = 10240
E = 320000
D = 128
NC, NS = 1, 16
NW = NC * NS
CH = 128
EP = 327680
EPT = EP // NW
NJ = EPT // CH
RPT = NP // NS
MB = NP // 8
_mesh = plsc.VectorSubcoreMesh(core_axis_name="c", subcore_axis_name="s",
                               num_cores=NC, num_subcores=NS)


PH = 4
HJ = NJ // PH


def _agg_body(hs_hbm, src_hbm, dst_hbm, zeros_hbm, sidx_hbm, part,
              sv, dv, rows0, rows1, sidx, acc, sem0, sem1):
    c = lax.axis_index("c")
    s = lax.axis_index("s")
    w = s * NC + c
    base = s * RPT
    pltpu.sync_copy(sidx_hbm.at[s], sidx)
    pltpu.sync_copy(zeros_hbm, rows0)
    for q in range(RPT // 128):
        pltpu.sync_copy(rows0, acc.at[sidx.at[q]])
    plsc.subcore_barrier()

    for p in range(PH):
        pltpu.sync_copy(src_hbm.at[w].at[pl.ds(p * HJ, HJ)], sv)
        pltpu.sync_copy(dst_hbm.at[w].at[pl.ds(p * HJ, HJ)], dv)
        pltpu.async_copy(hs_hbm.at[sv.at[0]], rows0, sem0)

        @pl.loop(0, HJ, step=2)
        def _(j):
            pltpu.async_copy(hs_hbm.at[sv.at[j + 1]], rows1, sem1)
            pltpu.make_async_copy(hs_hbm.at[sv.at[j]], rows0, sem0).wait()
            pltpu.sync_copy(rows0, acc.at[dv.at[j]], add=True)
            nxt = jnp.minimum(j + 2, HJ - 1)
            pltpu.async_copy(hs_hbm.at[sv.at[nxt]], rows0, sem0)
            pltpu.make_async_copy(hs_hbm.at[sv.at[j]], rows1, sem1).wait()
            pltpu.sync_copy(rows1, acc.at[dv.at[j + 1]], add=True)

        pltpu.make_async_copy(hs_hbm.at[sv.at[0]], rows0, sem0).wait()

    plsc.subcore_barrier()
    for q in range(RPT // 128):
        pltpu.async_copy(acc.at[sidx.at[q]], rows0, sem0).wait()
        pltpu.sync_copy(rows0, part.at[c].at[pl.ds(base + q * 128, 128)])


_agg_kernel = pl.kernel(
    _agg_body,
    out_type=jax.ShapeDtypeStruct((NC, NP, D), jnp.float32),
    mesh=_mesh,
    scratch_types=[
        pltpu.VMEM((HJ, CH), jnp.int32),
        pltpu.VMEM((HJ, CH), jnp.int32),
        pltpu.VMEM((CH, D), jnp.float32),
        pltpu.VMEM((CH, D), jnp.float32),
        pltpu.VMEM((RPT // 128, 128), jnp.int32),
        pltpu.VMEM_SHARED((NP, D), jnp.float32),
        pltpu.SemaphoreType.DMA,
        pltpu.SemaphoreType.DMA,
    ],
)


def _dis_body(degp_ref, out_ref):
    out_ref[...] = lax.rsqrt(1.0 + degp_ref[0])


def _dis_kernel(degp):
    return pl.pallas_call(
        _dis_body,
        grid=(NP // MB,),
        in_specs=[pl.BlockSpec((NC, MB, D), lambda i: (0, i, 0))],
        out_specs=pl.BlockSpec((MB, D), lambda i: (i, 0)),
        out_shape=jax.ShapeDtypeStruct((NP, D), jnp.float32),
    )(degp)


def _pre_body(x_ref, w_ref, dis_ref, out_ref):
    hw = jnp.dot(x_ref[...], w_ref[...], preferred_element_type=jnp.float32,
                 precision=lax.Precision.HIGHEST)
    out_ref[...] = hw * dis_ref[...]


def _pre_kernel(x, w, dis):
    return pl.pallas_call(
        _pre_body,
        grid=(NP // MB,),
        in_specs=[
            pl.BlockSpec((MB, D), lambda i: (i, 0)),
            pl.BlockSpec((D, D), lambda i: (0, 0)),
            pl.BlockSpec((MB, D), lambda i: (i, 0)),
        ],
        out_specs=pl.BlockSpec((MB, D), lambda i: (i, 0)),
        out_shape=jax.ShapeDtypeStruct((NP, D), jnp.float32),
    )(x, w, dis)


def _mid_body(part_ref, hs_ref, dis_ref, b_ref, w_ref, out_ref):
    agg = part_ref[0] + hs_ref[...]
    h1 = jax.nn.relu(dis_ref[...] * agg + b_ref[...])
    hw = jnp.dot(h1, w_ref[...], preferred_element_type=jnp.float32,
                 precision=lax.Precision.HIGHEST)
    out_ref[...] = hw * dis_ref[...]


def _mid_kernel(part, hs, dis, b, w):
    return pl.pallas_call(
        _mid_body,
        grid=(NP // MB,),
        in_specs=[
            pl.BlockSpec((NC, MB, D), lambda i: (0, i, 0)),
            pl.BlockSpec((MB, D), lambda i: (i, 0)),
            pl.BlockSpec((MB, D), lambda i: (i, 0)),
            pl.BlockSpec((1, D), lambda i: (0, 0)),
            pl.BlockSpec((D, D), lambda i: (0, 0)),
        ],
        out_specs=pl.BlockSpec((MB, D), lambda i: (i, 0)),
        out_shape=jax.ShapeDtypeStruct((NP, D), jnp.float32),
    )(part, hs, dis, b, w)


def _post_body(part_ref, hs_ref, dis_ref, b_ref, out_ref):
    agg = part_ref[0] + hs_ref[...]
    out_ref[...] = dis_ref[...] * agg + b_ref[...]


def _post_kernel(part, hs, dis, b):
    return pl.pallas_call(
        _post_body,
        grid=(NP // MB,),
        in_specs=[
            pl.BlockSpec((NC, MB, D), lambda i: (0, i, 0)),
            pl.BlockSpec((MB, D), lambda i: (i, 0)),
            pl.BlockSpec((MB, D), lambda i: (i, 0)),
            pl.BlockSpec((1, D), lambda i: (0, 0)),
        ],
        out_specs=pl.BlockSpec((MB, D), lambda i: (i, 0)),
        out_shape=jax.ShapeDtypeStruct((NP, D), jnp.float32),
    )(part, hs, dis, b)


@jax.jit
def kernel(x, edge_index, W1, b1, W2, b2):
    pad = jnp.full((EP - E,), N, jnp.int32)
    src = jnp.concatenate([edge_index[0], pad]).reshape(NW, NJ, CH)
    dst = jnp.concatenate([edge_index[1], pad]).reshape(NW, NJ, CH)
    xp = jnp.pad(x, ((0, NP - N), (0, 0)))
    zrows = jnp.zeros((128, D), jnp.float32)
    onesmat = jnp.ones((NP, D), jnp.float32)
    stripes = (jnp.arange(NS, dtype=jnp.int32)[:, None] * RPT
               + jnp.arange(RPT, dtype=jnp.int32)[None, :])
    sidx = stripes.reshape(NS, RPT // 128, 128)

    degp = _agg_kernel(onesmat, src, dst, zrows, sidx)
    dis = _dis_kernel(degp)

    hs1 = _pre_kernel(xp, W1, dis)
    part1 = _agg_kernel(hs1, src, dst, zrows, sidx)
    hs2 = _mid_kernel(part1, hs1, dis, b1.reshape(1, D), W2)
    part2 = _agg_kernel(hs2, src, dst, zrows, sidx)
    out = _post_kernel(part2, hs2, dis, b2.reshape(1, D))
    return out[:N]


# speedup vs baseline: 1.5792x; 1.5792x over previous
import jax
import jax.numpy as jnp
from jax import lax
from jax.experimental import pallas as pl
from jax.experimental.pallas import tpu as pltpu
from jax.experimental.pallas import tpu_sc as plsc

N = 10000
NP = 10240
E = 320000
D = 128
NC, NS = 2, 16
NW = NC * NS
CH = 128
EP = 327680
EPT = EP // NW
NJ = EPT // CH
RPT = NP // NS
MB = NP // 8
_mesh = plsc.VectorSubcoreMesh(core_axis_name="c", subcore_axis_name="s",
                               num_cores=NC, num_subcores=NS)


PH = 2
HJ = NJ // PH


def _agg_body(hs_hbm, src_hbm, dst_hbm, zeros_hbm, sidx_hbm, part,
              sv, dv, rows0, rows1, sidx, acc, sem0, sem1):
    c = lax.axis_index("c")
    s = lax.axis_index("s")
    w = s * NC + c
    base = s * RPT
    pltpu.sync_copy(sidx_hbm.at[s], sidx)
    pltpu.sync_copy(zeros_hbm, rows0)
    for q in range(RPT // 128):
        pltpu.sync_copy(rows0, acc.at[sidx.at[q]])
    plsc.subcore_barrier()

    for p in range(PH):
        pltpu.sync_copy(src_hbm.at[w].at[pl.ds(p * HJ, HJ)], sv)
        pltpu.sync_copy(dst_hbm.at[w].at[pl.ds(p * HJ, HJ)], dv)
        pltpu.async_copy(hs_hbm.at[sv.at[0]], rows0, sem0)

        @pl.loop(0, HJ, step=2)
        def _(j):
            pltpu.async_copy(hs_hbm.at[sv.at[j + 1]], rows1, sem1)
            pltpu.make_async_copy(hs_hbm.at[sv.at[j]], rows0, sem0).wait()
            pltpu.sync_copy(rows0, acc.at[dv.at[j]], add=True)
            nxt = jnp.minimum(j + 2, HJ - 1)
            pltpu.async_copy(hs_hbm.at[sv.at[nxt]], rows0, sem0)
            pltpu.make_async_copy(hs_hbm.at[sv.at[j]], rows1, sem1).wait()
            pltpu.sync_copy(rows1, acc.at[dv.at[j + 1]], add=True)

        pltpu.make_async_copy(hs_hbm.at[sv.at[0]], rows0, sem0).wait()

    plsc.subcore_barrier()
    for q in range(RPT // 128):
        pltpu.async_copy(acc.at[sidx.at[q]], rows0, sem0).wait()
        pltpu.sync_copy(rows0, part.at[c].at[pl.ds(base + q * 128, 128)])


_agg_kernel = pl.kernel(
    _agg_body,
    out_type=jax.ShapeDtypeStruct((NC, NP, D), jnp.float32),
    mesh=_mesh,
    scratch_types=[
        pltpu.VMEM((HJ, CH), jnp.int32),
        pltpu.VMEM((HJ, CH), jnp.int32),
        pltpu.VMEM((CH, D), jnp.float32),
        pltpu.VMEM((CH, D), jnp.float32),
        pltpu.VMEM((RPT // 128, 128), jnp.int32),
        pltpu.VMEM_SHARED((NP, D), jnp.float32),
        pltpu.SemaphoreType.DMA,
        pltpu.SemaphoreType.DMA,
    ],
)


def _deg_body(dst_hbm, ones_hbm, zeros_hbm, sidx_hbm, part,
              dv, rows0, ones_v, sidx, acc, sem0):
    c = lax.axis_index("c")
    s = lax.axis_index("s")
    w = s * NC + c
    base = s * RPT
    pltpu.sync_copy(sidx_hbm.at[s], sidx)
    pltpu.sync_copy(zeros_hbm, rows0)
    pltpu.sync_copy(ones_hbm, ones_v)
    for q in range(RPT // 128):
        pltpu.sync_copy(rows0, acc.at[sidx.at[q]])
    plsc.subcore_barrier()

    for p in range(PH):
        pltpu.sync_copy(dst_hbm.at[w].at[pl.ds(p * HJ, HJ)], dv)

        @pl.loop(0, HJ)
        def _(j):
            pltpu.sync_copy(ones_v, acc.at[dv.at[j]], add=True)

    plsc.subcore_barrier()
    for q in range(RPT // 128):
        pltpu.async_copy(acc.at[sidx.at[q]], rows0, sem0).wait()
        pltpu.sync_copy(rows0, part.at[c].at[pl.ds(base + q * 128, 128)])


_deg_kernel = pl.kernel(
    _deg_body,
    out_type=jax.ShapeDtypeStruct((NC, NP, D), jnp.float32),
    mesh=_mesh,
    scratch_types=[
        pltpu.VMEM((HJ, CH), jnp.int32),
        pltpu.VMEM((CH, D), jnp.float32),
        pltpu.VMEM((CH, D), jnp.float32),
        pltpu.VMEM((RPT // 128, 128), jnp.int32),
        pltpu.VMEM_SHARED((NP, D), jnp.float32),
        pltpu.SemaphoreType.DMA,
    ],
)


def _dis_body(degp_ref, out_ref):
    out_ref[...] = lax.rsqrt(1.0 + degp_ref[0] + degp_ref[1])


def _dis_kernel(degp):
    return pl.pallas_call(
        _dis_body,
        grid=(NP // MB,),
        in_specs=[pl.BlockSpec((NC, MB, D), lambda i: (0, i, 0))],
        out_specs=pl.BlockSpec((MB, D), lambda i: (i, 0)),
        out_shape=jax.ShapeDtypeStruct((NP, D), jnp.float32),
    )(degp)


def _pre_body(x_ref, w_ref, dis_ref, out_ref):
    hw = jnp.dot(x_ref[...], w_ref[...], preferred_element_type=jnp.float32,
                 precision=lax.Precision.HIGHEST)
    out_ref[...] = hw * dis_ref[...]


def _pre_kernel(x, w, dis):
    return pl.pallas_call(
        _pre_body,
        grid=(NP // MB,),
        in_specs=[
            pl.BlockSpec((MB, D), lambda i: (i, 0)),
            pl.BlockSpec((D, D), lambda i: (0, 0)),
            pl.BlockSpec((MB, D), lambda i: (i, 0)),
        ],
        out_specs=pl.BlockSpec((MB, D), lambda i: (i, 0)),
        out_shape=jax.ShapeDtypeStruct((NP, D), jnp.float32),
    )(x, w, dis)


def _mid_body(part_ref, hs_ref, dis_ref, b_ref, w_ref, out_ref):
    agg = part_ref[0] + part_ref[1] + hs_ref[...]
    h1 = jax.nn.relu(dis_ref[...] * agg + b_ref[...])
    hw = jnp.dot(h1, w_ref[...], preferred_element_type=jnp.float32,
                 precision=lax.Precision.HIGHEST)
    out_ref[...] = hw * dis_ref[...]


def _mid_kernel(part, hs, dis, b, w):
    return pl.pallas_call(
        _mid_body,
        grid=(NP // MB,),
        in_specs=[
            pl.BlockSpec((NC, MB, D), lambda i: (0, i, 0)),
            pl.BlockSpec((MB, D), lambda i: (i, 0)),
            pl.BlockSpec((MB, D), lambda i: (i, 0)),
            pl.BlockSpec((1, D), lambda i: (0, 0)),
            pl.BlockSpec((D, D), lambda i: (0, 0)),
        ],
        out_specs=pl.BlockSpec((MB, D), lambda i: (i, 0)),
        out_shape=jax.ShapeDtypeStruct((NP, D), jnp.float32),
    )(part, hs, dis, b, w)


def _post_body(part_ref, hs_ref, dis_ref, b_ref, out_ref):
    agg = part_ref[0] + part_ref[1] + hs_ref[...]
    out_ref[...] = dis_ref[...] * agg + b_ref[...]


def _post_kernel(part, hs, dis, b):
    return pl.pallas_call(
        _post_body,
        grid=(NP // MB,),
        in_specs=[
            pl.BlockSpec((NC, MB, D), lambda i: (0, i, 0)),
            pl.BlockSpec((MB, D), lambda i: (i, 0)),
            pl.BlockSpec((MB, D), lambda i: (i, 0)),
            pl.BlockSpec((1, D), lambda i: (0, 0)),
        ],
        out_specs=pl.BlockSpec((MB, D), lambda i: (i, 0)),
        out_shape=jax.ShapeDtypeStruct((NP, D), jnp.float32),
    )(part, hs, dis, b)


@jax.jit
def kernel(x, edge_index, W1, b1, W2, b2):
    pad = jnp.full((EP - E,), N, jnp.int32)
    src = jnp.concatenate([edge_index[0], pad]).reshape(NW, NJ, CH)
    dst = jnp.concatenate([edge_index[1], pad]).reshape(NW, NJ, CH)
    xp = jnp.pad(x, ((0, NP - N), (0, 0)))
    zrows = jnp.zeros((128, D), jnp.float32)
    onesrows = jnp.ones((CH, D), jnp.float32)
    stripes = (jnp.arange(NS, dtype=jnp.int32)[:, None] * RPT
               + jnp.arange(RPT, dtype=jnp.int32)[None, :])
    sidx = stripes.reshape(NS, RPT // 128, 128)

    degp = _deg_kernel(dst, onesrows, zrows, sidx)
    dis = _dis_kernel(degp)

    hs1 = _pre_kernel(xp, W1, dis)
    part1 = _agg_kernel(hs1, src, dst, zrows, sidx)
    hs2 = _mid_kernel(part1, hs1, dis, b1.reshape(1, D), W2)
    part2 = _agg_kernel(hs2, src, dst, zrows, sidx)
    out = _post_kernel(part2, hs2, dis, b2.reshape(1, D))
    return out[:N]
